# baseline (speedup 1.0000x reference)
import functools

import jax
import jax.numpy as jnp
from jax import lax
from jax.experimental import pallas as pl
from jax.experimental.pallas import tpu as pltpu
from jax.experimental.pallas import tpu_sc as plsc

B = 16384
NW = 32
BPW = B // NW

MF_D = 8
W = 32

MF_PR = BPW * MF_D // 128
MF_OR = B * MF_D // 128
MLP_PR = BPW * 16 // 128
MLP_OR = B * 16 // 128


def _sc_gather(user2d, item2d, mfu_t, mfi_t, mlpu_t, mlpi_t):
    mesh = plsc.VectorSubcoreMesh(core_axis_name="c", subcore_axis_name="s")

    @functools.partial(
        pl.kernel,
        mesh=mesh,
        compiler_params=pltpu.CompilerParams(needs_layout_passes=False),
        out_type=[
            jax.ShapeDtypeStruct((MF_OR, 128), jnp.float32),
            jax.ShapeDtypeStruct((MLP_OR, 128), jnp.int32),
            jax.ShapeDtypeStruct((MLP_OR, 128), jnp.int32),
        ],
        scratch_types=[
            pltpu.VMEM((BPW,), jnp.int32),
            pltpu.VMEM((BPW,), jnp.int32),
            pltpu.VMEM((64, W), jnp.float32),
            pltpu.VMEM((64, W), jnp.float32),
            pltpu.VMEM((64, 128), jnp.float32),
            pltpu.VMEM((64, 128), jnp.float32),
            pltpu.VMEM((4, 128), jnp.float32),
            pltpu.VMEM((8, 128), jnp.int32),
            pltpu.SemaphoreType.DMA,
        ],
    )
    def k(u_hbm, i_hbm, mfu_tr, mfi_tr, mlpu_tr, mlpi_tr,
          mf_o, mlpu_o, mlpi_o,
          uidx, iidx, mfu_v, mfi_v, mlpu_v, mlpi_v,
          mf_p, mlp_p, sem):
        wid = lax.axis_index("s") * 2 + lax.axis_index("c")
        pltpu.sync_copy(u_hbm.at[wid], uidx)
        pltpu.sync_copy(i_hbm.at[wid], iidx)
        lane = lax.iota(jnp.int32, 16)
        lane8 = lane & 7
        half = lax.shift_right_logical(lane, 3)
        HC = 64
        NCHK = BPW // HC
        MFH = HC * MF_D // 128
        MLH = HC * 16 // 128

        for hh in range(NCHK):
            def body(v, _):
                uvec = uidx[pl.ds(hh * HC + v * 16, 16)]
                ivec = iidx[pl.ds(hh * HC + v * 16, 16)]
                for l in range(16):
                    s = v * 16 + l
                    u = uvec[l]
                    i = ivec[l]
                    uq = lax.shift_right_logical(u, 2)
                    iq = lax.shift_right_logical(i, 2)
                    ul = lax.shift_right_logical(u, 2)
                    il = lax.shift_right_logical(i, 2)
                    pltpu.async_copy(
                        mfu_tr.at[pl.ds(uq, 1)], mfu_v.at[pl.ds(s, 1)], sem)
                    pltpu.async_copy(
                        mlpu_tr.at[pl.ds(ul, 1)], mlpu_v.at[pl.ds(s, 1)], sem)
                    pltpu.async_copy(
                        mfi_tr.at[pl.ds(iq, 1)], mfi_v.at[pl.ds(s, 1)], sem)
                    pltpu.async_copy(
                        mlpi_tr.at[pl.ds(il, 1)], mlpi_v.at[pl.ds(s, 1)], sem)
                return ()

            lax.fori_loop(0, HC // 16, body, (), unroll=False)
            pltpu.make_async_copy(mfu_tr.at[pl.ds(0, 64)], mfu_v, sem).wait()
            pltpu.make_async_copy(mfi_tr.at[pl.ds(0, 64)], mfi_v, sem).wait()
            pltpu.make_async_copy(
                mlpu_tr.at[pl.ds(0, 64)], mlpu_v, sem).wait()
            pltpu.make_async_copy(
                mlpi_tr.at[pl.ds(0, 64)], mlpi_v, sem).wait()

            def mfrow(r, _):
                for q in range(8):
                    p = r * 8 + q
                    rowv = 2 * p + half
                    gidx = hh * HC + rowv
                    ucolv = (plsc.load_gather(uidx, [gidx]) & 3) * MF_D + lane8
                    icolv = (plsc.load_gather(iidx, [gidx]) & 3) * MF_D + lane8
                    uvals = plsc.load_gather(mfu_v, [rowv, ucolv])
                    ivals = plsc.load_gather(mfi_v, [rowv, icolv])
                    mf_p[r, pl.ds(q * 16, 16)] = uvals * ivals
                return ()

            lax.fori_loop(0, MFH, mfrow, (), unroll=False)
            pltpu.sync_copy(
                mf_p, mf_o.at[pl.ds(wid * MF_PR + hh * MFH, MFH)])

            for buf, ibuf, out in (
                    (mlpu_v, uidx, mlpu_o), (mlpi_v, iidx, mlpi_o)):
                def prow(v, _):
                    xvec = ibuf[pl.ds(hh * HC + v * 16, 16)]
                    for l in range(16):
                        s = v * 16 + l
                        c32 = (xvec[l] & 3) * W
                        a = buf[s, pl.ds(c32, 16)]
                        b = buf[s, pl.ds(c32 + 16, 16)]
                        w = plsc.bitcast(
                            plsc.pack(
                                a, b, format=plsc.PackFormat.INTERLEAVED),
                            jnp.int32)
                        mlp_p[2 * v + l // 8, pl.ds((l % 8) * 16, 16)] = w
                    return ()
                lax.fori_loop(0, HC // 16, prow, (), unroll=False)
                pltpu.sync_copy(
                    mlp_p, out.at[pl.ds(wid * MLP_PR + hh * MLH, MLH)])

    return k(user2d, item2d, mfu_t, mfi_t, mlpu_t, mlpi_t)


def _unpack2(w):
    a = lax.bitcast_convert_type(lax.shift_left(w, 16), jnp.float32)
    b = lax.bitcast_convert_type(w & jnp.int32(-65536), jnp.float32)
    return a, b


def _tc_body(mf_r, mlpu_r, mlpi_r,
             w0_r, b0_r, w1_r, b1_r, w2_r, b2_r, wp_r, bp_r, o_r):
    w0 = w0_r[...]
    ua, ub = _unpack2(mlpu_r[...])
    ia, ib = _unpack2(mlpi_r[...])
    h = jnp.dot(ua, w0[0:16, :], preferred_element_type=jnp.float32)
    h = h + jnp.dot(ub, w0[16:32, :], preferred_element_type=jnp.float32)
    h = h + jnp.dot(ia, w0[32:48, :], preferred_element_type=jnp.float32)
    h = h + jnp.dot(ib, w0[48:64, :], preferred_element_type=jnp.float32)
    h = jnp.maximum(h + b0_r[...], 0.0)
    h = jnp.maximum(
        jnp.dot(h, w1_r[...], preferred_element_type=jnp.float32) + b1_r[...], 0.0)
    h = jnp.maximum(
        jnp.dot(h, w2_r[...], preferred_element_type=jnp.float32) + b2_r[...], 0.0)
    wp = wp_r[...]
    p = jnp.dot(mf_r[...], wp[:MF_D, :],
                preferred_element_type=jnp.float32)
    p = p + jnp.dot(h, wp[MF_D:, :], preferred_element_type=jnp.float32)
    o_r[...] = p + bp_r[...]


def _tc_mlp(mf, mlpu, mlpi, W0, b0, W1, b1, W2, b2, Wp, bp):
    BLK = 2048
    grid = (B // BLK,)

    def full(shape):
        return pl.BlockSpec(shape, lambda i: (0,) * len(shape))

    return pl.pallas_call(
        _tc_body,
        grid=grid,
        in_specs=[
            pl.BlockSpec((BLK, MF_D), lambda i: (i, 0)),
            pl.BlockSpec((BLK, 16), lambda i: (i, 0)),
            pl.BlockSpec((BLK, 16), lambda i: (i, 0)),
            full(W0.shape), full(b0.shape), full(W1.shape), full(b1.shape),
            full(W2.shape), full(b2.shape), full(Wp.shape), full(bp.shape),
        ],
        out_specs=pl.BlockSpec((BLK, 1), lambda i: (i, 0)),
        out_shape=jax.ShapeDtypeStruct((B, 1), jnp.float32),
    )(mf, mlpu, mlpi, W0, b0, W1, b1, W2, b2, Wp, bp)


def kernel(user, item, mf_emb_user, mf_emb_item, mlp_emb_user, mlp_emb_item,
           W0, b0, W1, b1, W2, b2, Wp, bp):
    user = user.astype(jnp.int32)
    item = item.astype(jnp.int32)
    u2 = user.reshape(NW, BPW)
    i2 = item.reshape(NW, BPW)
    mf, mlpu, mlpi = _sc_gather(
        u2, i2,
        mf_emb_user.reshape(-1, W), mf_emb_item.reshape(-1, W),
        mlp_emb_user.reshape(-1, 128), mlp_emb_item.reshape(-1, 128))
    mf = mf.reshape(B, MF_D)
    mlpu = mlpu.reshape(B, 16)
    mlpi = mlpi.reshape(B, 16)
    return _tc_mlp(
        mf, mlpu, mlpi,
        W0, b0.reshape(1, -1), W1, b1.reshape(1, -1),
        W2, b2.reshape(1, -1), Wp, bp.reshape(1, 1))


# speedup vs baseline: 1.1361x; 1.1361x over previous
import functools

import jax
import jax.numpy as jnp
from jax import lax
from jax.experimental import pallas as pl
from jax.experimental.pallas import tpu as pltpu
from jax.experimental.pallas import tpu_sc as plsc

B = 16384
NW = 32
BPW = B // NW

MF_D = 8
W = 32

MF_PR = BPW * MF_D // 128
MF_OR = B * MF_D // 128
MLP_PR = BPW * 16 // 128
MLP_OR = B * 16 // 128


def _sc_gather(user2d, item2d, mfu_t, mfi_t, mlpu_t, mlpi_t):
    mesh = plsc.VectorSubcoreMesh(core_axis_name="c", subcore_axis_name="s")

    @functools.partial(
        pl.kernel,
        mesh=mesh,
        compiler_params=pltpu.CompilerParams(needs_layout_passes=False),
        out_type=[
            jax.ShapeDtypeStruct((MF_OR, 128), jnp.float32),
            jax.ShapeDtypeStruct((MLP_OR, 128), jnp.int32),
            jax.ShapeDtypeStruct((MLP_OR, 128), jnp.int32),
        ],
        scratch_types=[
            pltpu.VMEM((BPW,), jnp.int32),
            pltpu.VMEM((BPW,), jnp.int32),
            pltpu.VMEM((64, W), jnp.float32),
            pltpu.VMEM((64, W), jnp.float32),
            pltpu.VMEM((64, W), jnp.float32),
            pltpu.VMEM((64, 128), jnp.float32),
            pltpu.VMEM((4, 128), jnp.float32),
            pltpu.VMEM((8, 128), jnp.int32),
            pltpu.SemaphoreType.DMA,
        ],
    )
    def k(u_hbm, i_hbm, mfu_tr, mfi_tr, mlpu_tr, mlpi_tr,
          mf_o, mlpu_o, mlpi_o,
          uidx, iidx, mfu_v, mfi_v, mlpu_v, mlpi_v,
          mf_p, mlp_p, sem):
        wid = lax.axis_index("s") * 2 + lax.axis_index("c")
        pltpu.sync_copy(u_hbm.at[wid], uidx)
        pltpu.sync_copy(i_hbm.at[wid], iidx)
        lane = lax.iota(jnp.int32, 16)
        lane8 = lane & 7
        half = lax.shift_right_logical(lane, 3)
        HC = 64
        NCHK = BPW // HC
        MFH = HC * MF_D // 128
        MLH = HC * 16 // 128

        for hh in range(NCHK):
            def body(v, _):
                uvec = uidx[pl.ds(hh * HC + v * 16, 16)]
                ivec = iidx[pl.ds(hh * HC + v * 16, 16)]
                for l in range(16):
                    s = v * 16 + l
                    u = uvec[l]
                    i = ivec[l]
                    uq = lax.shift_right_logical(u, 2)
                    iq = lax.shift_right_logical(i, 2)
                    il = lax.shift_right_logical(i, 2)
                    pltpu.async_copy(
                        mfu_tr.at[pl.ds(uq, 1)], mfu_v.at[pl.ds(s, 1)], sem)
                    pltpu.async_copy(
                        mlpu_tr.at[pl.ds(u, 1)], mlpu_v.at[pl.ds(s, 1)], sem)
                    pltpu.async_copy(
                        mfi_tr.at[pl.ds(iq, 1)], mfi_v.at[pl.ds(s, 1)], sem)
                    pltpu.async_copy(
                        mlpi_tr.at[pl.ds(il, 1)], mlpi_v.at[pl.ds(s, 1)], sem)
                return ()

            lax.fori_loop(0, HC // 16, body, (), unroll=False)
            pltpu.make_async_copy(mfu_tr.at[pl.ds(0, 64)], mfu_v, sem).wait()
            pltpu.make_async_copy(mfi_tr.at[pl.ds(0, 64)], mfi_v, sem).wait()
            pltpu.make_async_copy(
                mlpu_tr.at[pl.ds(0, 64)], mlpu_v, sem).wait()
            pltpu.make_async_copy(
                mlpi_tr.at[pl.ds(0, 64)], mlpi_v, sem).wait()

            def mfrow(r, _):
                for q in range(8):
                    p = r * 8 + q
                    rowv = 2 * p + half
                    gidx = hh * HC + rowv
                    ucolv = (plsc.load_gather(uidx, [gidx]) & 3) * MF_D + lane8
                    icolv = (plsc.load_gather(iidx, [gidx]) & 3) * MF_D + lane8
                    uvals = plsc.load_gather(mfu_v, [rowv, ucolv])
                    ivals = plsc.load_gather(mfi_v, [rowv, icolv])
                    mf_p[r, pl.ds(q * 16, 16)] = uvals * ivals
                return ()

            lax.fori_loop(0, MFH, mfrow, (), unroll=False)
            pltpu.sync_copy(
                mf_p, mf_o.at[pl.ds(wid * MF_PR + hh * MFH, MFH)])

            def prow_u(v, _):
                for l in range(16):
                    s = v * 16 + l
                    a = mlpu_v[s, pl.ds(0, 16)]
                    b = mlpu_v[s, pl.ds(16, 16)]
                    w = plsc.bitcast(
                        plsc.pack(a, b, format=plsc.PackFormat.INTERLEAVED),
                        jnp.int32)
                    mlp_p[2 * v + l // 8, pl.ds((l % 8) * 16, 16)] = w
                return ()

            lax.fori_loop(0, HC // 16, prow_u, (), unroll=False)
            pltpu.sync_copy(
                mlp_p, mlpu_o.at[pl.ds(wid * MLP_PR + hh * MLH, MLH)])

            def prow_i(v, _):
                xvec = iidx[pl.ds(hh * HC + v * 16, 16)]
                for l in range(16):
                    s = v * 16 + l
                    c32 = (xvec[l] & 3) * W
                    a = mlpi_v[s, pl.ds(c32, 16)]
                    b = mlpi_v[s, pl.ds(c32 + 16, 16)]
                    w = plsc.bitcast(
                        plsc.pack(a, b, format=plsc.PackFormat.INTERLEAVED),
                        jnp.int32)
                    mlp_p[2 * v + l // 8, pl.ds((l % 8) * 16, 16)] = w
                return ()

            lax.fori_loop(0, HC // 16, prow_i, (), unroll=False)
            pltpu.sync_copy(
                mlp_p, mlpi_o.at[pl.ds(wid * MLP_PR + hh * MLH, MLH)])

    return k(user2d, item2d, mfu_t, mfi_t, mlpu_t, mlpi_t)


def _unpack2(w):
    a = lax.bitcast_convert_type(lax.shift_left(w, 16), jnp.float32)
    b = lax.bitcast_convert_type(w & jnp.int32(-65536), jnp.float32)
    return a, b


def _tc_body(mf_r, mlpu_r, mlpi_r,
             w0_r, b0_r, w1_r, b1_r, w2_r, b2_r, wp_r, bp_r, o_r):
    w0 = w0_r[...]
    ua, ub = _unpack2(mlpu_r[...])
    ia, ib = _unpack2(mlpi_r[...])
    h = jnp.dot(ua, w0[0:16, :], preferred_element_type=jnp.float32)
    h = h + jnp.dot(ub, w0[16:32, :], preferred_element_type=jnp.float32)
    h = h + jnp.dot(ia, w0[32:48, :], preferred_element_type=jnp.float32)
    h = h + jnp.dot(ib, w0[48:64, :], preferred_element_type=jnp.float32)
    h = jnp.maximum(h + b0_r[...], 0.0)
    h = jnp.maximum(
        jnp.dot(h, w1_r[...], preferred_element_type=jnp.float32) + b1_r[...], 0.0)
    h = jnp.maximum(
        jnp.dot(h, w2_r[...], preferred_element_type=jnp.float32) + b2_r[...], 0.0)
    wp = wp_r[...]
    p = jnp.dot(mf_r[...], wp[:MF_D, :],
                preferred_element_type=jnp.float32)
    p = p + jnp.dot(h, wp[MF_D:, :], preferred_element_type=jnp.float32)
    o_r[...] = p + bp_r[...]


def _tc_mlp(mf, mlpu, mlpi, W0, b0, W1, b1, W2, b2, Wp, bp):
    BLK = 2048
    grid = (B // BLK,)

    def full(shape):
        return pl.BlockSpec(shape, lambda i: (0,) * len(shape))

    return pl.pallas_call(
        _tc_body,
        grid=grid,
        in_specs=[
            pl.BlockSpec((BLK, MF_D), lambda i: (i, 0)),
            pl.BlockSpec((BLK, 16), lambda i: (i, 0)),
            pl.BlockSpec((BLK, 16), lambda i: (i, 0)),
            full(W0.shape), full(b0.shape), full(W1.shape), full(b1.shape),
            full(W2.shape), full(b2.shape), full(Wp.shape), full(bp.shape),
        ],
        out_specs=pl.BlockSpec((BLK, 1), lambda i: (i, 0)),
        out_shape=jax.ShapeDtypeStruct((B, 1), jnp.float32),
    )(mf, mlpu, mlpi, W0, b0, W1, b1, W2, b2, Wp, bp)


def kernel(user, item, mf_emb_user, mf_emb_item, mlp_emb_user, mlp_emb_item,
           W0, b0, W1, b1, W2, b2, Wp, bp):
    user = user.astype(jnp.int32)
    item = item.astype(jnp.int32)
    u2 = user.reshape(NW, BPW)
    i2 = item.reshape(NW, BPW)
    mf, mlpu, mlpi = _sc_gather(
        u2, i2,
        mf_emb_user.reshape(-1, W), mf_emb_item.reshape(-1, W),
        mlp_emb_user, mlp_emb_item.reshape(-1, 128))
    mf = mf.reshape(B, MF_D)
    mlpu = mlpu.reshape(B, 16)
    mlpi = mlpi.reshape(B, 16)
    return _tc_mlp(
        mf, mlpu, mlpi,
        W0, b0.reshape(1, -1), W1, b1.reshape(1, -1),
        W2, b2.reshape(1, -1), Wp, bp.reshape(1, 1))


# speedup vs baseline: 1.2246x; 1.0779x over previous
import functools

import jax
import jax.numpy as jnp
from jax import lax
from jax.experimental import pallas as pl
from jax.experimental.pallas import tpu as pltpu
from jax.experimental.pallas import tpu_sc as plsc

B = 16384
NW = 32
BPW = B // NW

MF_D = 8
W = 32

MF_PR = BPW * MF_D // 128
MF_OR = B * MF_D // 128
MLP_PR = BPW * 16 // 128
MLP_OR = B * 16 // 128


def _sc_gather(user2d, item2d, mfu_t, mfi_t, mlpu_t, mlpi_t):
    mesh = plsc.VectorSubcoreMesh(core_axis_name="c", subcore_axis_name="s")

    @functools.partial(
        pl.kernel,
        mesh=mesh,
        compiler_params=pltpu.CompilerParams(needs_layout_passes=False),
        out_type=[
            jax.ShapeDtypeStruct((MF_OR, 128), jnp.float32),
            jax.ShapeDtypeStruct((MLP_OR, 128), jnp.int32),
            jax.ShapeDtypeStruct((MLP_OR, 128), jnp.int32),
        ],
        scratch_types=[
            pltpu.VMEM((BPW,), jnp.int32),
            pltpu.VMEM((BPW,), jnp.int32),
            pltpu.VMEM((BPW // 4, W), jnp.float32),
            pltpu.VMEM((BPW // 4, W), jnp.float32),
            pltpu.VMEM((BPW // 4, W), jnp.float32),
            pltpu.VMEM((BPW // 4, W), jnp.float32),
            pltpu.VMEM((MF_PR // 4, 128), jnp.float32),
            pltpu.VMEM((MLP_PR // 4, 128), jnp.int32),
            pltpu.SemaphoreType.DMA,
        ],
    )
    def k(u_hbm, i_hbm, mfu_tr, mfi_tr, mlpu_tr, mlpi_tr,
          mf_o, mlpu_o, mlpi_o,
          uidx, iidx, mfu_v, mfi_v, mlpu_v, mlpi_v,
          mf_p, mlp_p, sem):
        wid = lax.axis_index("s") * 2 + lax.axis_index("c")
        pltpu.sync_copy(u_hbm.at[wid], uidx)
        pltpu.sync_copy(i_hbm.at[wid], iidx)
        lane = lax.iota(jnp.int32, 16)
        lane8 = lane & 7
        half = lax.shift_right_logical(lane, 3)
        HC = BPW // 4
        MFH = MF_PR // 4
        MLH = MLP_PR // 4

        for hh in range(4):
            def body(v, _):
                uvec = uidx[pl.ds(hh * HC + v * 16, 16)]
                ivec = iidx[pl.ds(hh * HC + v * 16, 16)]
                for l in range(16):
                    s = v * 16 + l
                    u = uvec[l]
                    i = ivec[l]
                    uq = lax.shift_right_logical(u, 2)
                    iq = lax.shift_right_logical(i, 2)
                    pltpu.async_copy(
                        mfu_tr.at[pl.ds(uq, 1)], mfu_v.at[pl.ds(s, 1)], sem)
                    pltpu.async_copy(
                        mlpu_tr.at[pl.ds(u, 1)], mlpu_v.at[pl.ds(s, 1)], sem)
                    pltpu.async_copy(
                        mfi_tr.at[pl.ds(iq, 1)], mfi_v.at[pl.ds(s, 1)], sem)
                    pltpu.async_copy(
                        mlpi_tr.at[pl.ds(i, 1)], mlpi_v.at[pl.ds(s, 1)], sem)
                return ()

            lax.fori_loop(0, HC // 16, body, (), unroll=False)
            pltpu.make_async_copy(mfu_tr.at[pl.ds(0, HC)], mfu_v, sem).wait()
            pltpu.make_async_copy(mfi_tr.at[pl.ds(0, HC)], mfi_v, sem).wait()
            pltpu.make_async_copy(mlpu_tr.at[pl.ds(0, HC)], mlpu_v, sem).wait()
            pltpu.make_async_copy(mlpi_tr.at[pl.ds(0, HC)], mlpi_v, sem).wait()

            def mfrow(r, _):
                for q in range(8):
                    p = r * 8 + q
                    rowv = 2 * p + half
                    gidx = hh * HC + rowv
                    ucolv = (plsc.load_gather(uidx, [gidx]) & 3) * MF_D + lane8
                    icolv = (plsc.load_gather(iidx, [gidx]) & 3) * MF_D + lane8
                    uvals = plsc.load_gather(mfu_v, [rowv, ucolv])
                    ivals = plsc.load_gather(mfi_v, [rowv, icolv])
                    mf_p[r, pl.ds(q * 16, 16)] = uvals * ivals
                return ()

            lax.fori_loop(0, MFH, mfrow, (), unroll=False)
            pltpu.sync_copy(
                mf_p, mf_o.at[pl.ds(wid * MF_PR + hh * MFH, MFH)])

            def mlp_pack(buf, dst):
                def prow(r, _):
                    for q in range(8):
                        s = r * 8 + q
                        a = buf[s, pl.ds(0, 16)]
                        b = buf[s, pl.ds(16, 16)]
                        w = plsc.bitcast(
                            plsc.pack(
                                a, b, format=plsc.PackFormat.INTERLEAVED),
                            jnp.int32)
                        dst[r, pl.ds(q * 16, 16)] = w
                    return ()
                lax.fori_loop(0, MLH, prow, (), unroll=False)

            mlp_pack(mlpu_v, mlp_p)
            pltpu.sync_copy(
                mlp_p, mlpu_o.at[pl.ds(wid * MLP_PR + hh * MLH, MLH)])
            mlp_pack(mlpi_v, mlp_p)
            pltpu.sync_copy(
                mlp_p, mlpi_o.at[pl.ds(wid * MLP_PR + hh * MLH, MLH)])

    return k(user2d, item2d, mfu_t, mfi_t, mlpu_t, mlpi_t)


def _unpack2(w):
    a = lax.bitcast_convert_type(lax.shift_left(w, 16), jnp.float32)
    b = lax.bitcast_convert_type(w & jnp.int32(-65536), jnp.float32)
    return a, b


def _tc_body(mf_r, mlpu_r, mlpi_r,
             w0_r, b0_r, w1_r, b1_r, w2_r, b2_r, wp_r, bp_r, o_r):
    w0 = w0_r[...]
    ua, ub = _unpack2(mlpu_r[...])
    ia, ib = _unpack2(mlpi_r[...])
    h = jnp.dot(ua, w0[0:16, :], preferred_element_type=jnp.float32)
    h = h + jnp.dot(ub, w0[16:32, :], preferred_element_type=jnp.float32)
    h = h + jnp.dot(ia, w0[32:48, :], preferred_element_type=jnp.float32)
    h = h + jnp.dot(ib, w0[48:64, :], preferred_element_type=jnp.float32)
    h = jnp.maximum(h + b0_r[...], 0.0)
    h = jnp.maximum(
        jnp.dot(h, w1_r[...], preferred_element_type=jnp.float32) + b1_r[...], 0.0)
    h = jnp.maximum(
        jnp.dot(h, w2_r[...], preferred_element_type=jnp.float32) + b2_r[...], 0.0)
    wp = wp_r[...]
    p = jnp.dot(mf_r[...], wp[:MF_D, :],
                preferred_element_type=jnp.float32)
    p = p + jnp.dot(h, wp[MF_D:, :], preferred_element_type=jnp.float32)
    o_r[...] = p + bp_r[...]


def _tc_mlp(mf, mlpu, mlpi, W0, b0, W1, b1, W2, b2, Wp, bp):
    BLK = 2048
    grid = (B // BLK,)

    def full(shape):
        return pl.BlockSpec(shape, lambda i: (0,) * len(shape))

    return pl.pallas_call(
        _tc_body,
        grid=grid,
        in_specs=[
            pl.BlockSpec((BLK, MF_D), lambda i: (i, 0)),
            pl.BlockSpec((BLK, 16), lambda i: (i, 0)),
            pl.BlockSpec((BLK, 16), lambda i: (i, 0)),
            full(W0.shape), full(b0.shape), full(W1.shape), full(b1.shape),
            full(W2.shape), full(b2.shape), full(Wp.shape), full(bp.shape),
        ],
        out_specs=pl.BlockSpec((BLK, 1), lambda i: (i, 0)),
        out_shape=jax.ShapeDtypeStruct((B, 1), jnp.float32),
    )(mf, mlpu, mlpi, W0, b0, W1, b1, W2, b2, Wp, bp)


def kernel(user, item, mf_emb_user, mf_emb_item, mlp_emb_user, mlp_emb_item,
           W0, b0, W1, b1, W2, b2, Wp, bp):
    user = user.astype(jnp.int32)
    item = item.astype(jnp.int32)
    u2 = user.reshape(NW, BPW)
    i2 = item.reshape(NW, BPW)
    mf, mlpu, mlpi = _sc_gather(
        u2, i2,
        mf_emb_user.reshape(-1, W), mf_emb_item.reshape(-1, W),
        mlp_emb_user, mlp_emb_item)
    mf = mf.reshape(B, MF_D)
    mlpu = mlpu.reshape(B, 16)
    mlpi = mlpi.reshape(B, 16)
    return _tc_mlp(
        mf, mlpu, mlpi,
        W0, b0.reshape(1, -1), W1, b1.reshape(1, -1),
        W2, b2.reshape(1, -1), Wp, bp.reshape(1, 1))
